# baseline (speedup 1.0000x reference)
import jax
import jax.numpy as jnp
from jax import lax
from jax.experimental import pallas as pl
from jax.experimental.pallas import tpu as pltpu
from jax.experimental.pallas import tpu_sc as plsc

VOCAB = 1_000_000
L = 200
D = 32
B = 4096

NC, NS = 2, 16
NW = NC * NS
BW = B // NW

LH, LL = L // 8, 8
DH, DL = D // 8, 8
PL = 4
G = L // PL
NBUF = 2
BWP = BW + 1


def _emb_body(x4_hbm, tok_hbm, pos_hbm, out_hbm, idx_v, rows_v, tblk_v, pos_v, gsem, wsem):
    cid = lax.axis_index("c")
    sid = lax.axis_index("s")
    wid = sid * NC + cid

    pltpu.sync_copy(pos_hbm, pos_v)

    iota = lax.iota(jnp.int32, 16)

    def fire(g):
        p = lax.rem(g, NBUF)
        lh = lax.div(g, LL // PL)
        ll0 = lax.rem(g, LL // PL) * PL
        pltpu.sync_copy(x4_hbm.at[lh, wid, pl.ds(ll0, PL)], idx_v.at[p])
        for j in range(PL):
            pltpu.async_copy(
                tok_hbm.at[idx_v.at[p, j]], rows_v.at[p, j], gsem.at[p]
            )

    fire(0)

    def chunk(g, _):
        p = lax.rem(g, NBUF)

        @pl.when(g + 1 < G)
        def _fire_next():
            fire(g + 1)

        for j in range(PL):
            pltpu.make_async_copy(
                tok_hbm.at[pl.ds(0, BW)], rows_v.at[p, j], gsem.at[p]
            ).wait()

        @pl.when(g >= NBUF)
        def _wait_writeback():
            for j in range(PL):
                for dh in range(DH):
                    pltpu.make_async_copy(
                        tblk_v.at[p, j, pl.ds(dh * DL, DL), pl.ds(0, BW)],
                        out_hbm.at[0, dh, 0],
                        wsem.at[p],
                    ).wait()

        for j in range(PL):
            lpos = g * PL + j
            prow0 = pos_v[lpos, pl.ds(0, 16)]
            prow1 = pos_v[lpos, pl.ds(16, 16)]
            rows = rows_v.at[p, j]
            tblk = tblk_v.at[p, j]

            @plsc.parallel_loop(0, BW, unroll=8)
            def _transpose(b):
                bsplat = jnp.broadcast_to(b, (16,))
                v0 = rows[b, pl.ds(0, 16)] + prow0
                v1 = rows[b, pl.ds(16, 16)] + prow1
                plsc.store_scatter(tblk, [iota, bsplat], v0)
                plsc.store_scatter(tblk, [iota + 16, bsplat], v1)

        for j in range(PL):
            for dh in range(DH):
                pltpu.async_copy(
                    tblk_v.at[p, j, pl.ds(dh * DL, DL), pl.ds(0, BW)],
                    out_hbm.at[g * PL + j, dh, wid],
                    wsem.at[p],
                )
        return 0

    lax.fori_loop(0, G, chunk, 0)

    for b in range(NBUF):
        for j in range(PL):
            for dh in range(DH):
                pltpu.make_async_copy(
                    tblk_v.at[b, j, pl.ds(dh * DL, DL), pl.ds(0, BW)],
                    out_hbm.at[0, dh, 0],
                    wsem.at[b],
                ).wait()


def _detile_body(in_ref, out_ref, scr):
    scr[:] = in_ref[:].T
    for q in range(4):
        out_ref[:, 32 * q : 32 * (q + 1)] = scr[pl.Slice(q, 128, 4), :]


def _detile(tt_T):
    return pl.pallas_call(
        _detile_body,
        grid=(VOCAB // 512,),
        in_specs=[pl.BlockSpec((D, 512), lambda k: (0, k))],
        out_specs=pl.BlockSpec((128, 128), lambda k: (k, 0)),
        out_shape=jax.ShapeDtypeStruct((VOCAB // 4, 128), jnp.float32),
        scratch_shapes=[pltpu.VMEM((512, D), jnp.float32)],
        compiler_params=pltpu.CompilerParams(
            dimension_semantics=("arbitrary",)
        ),
    )(tt_T)


@jax.jit
def _emb(x4, token_table, pos_table):
    tlin = _detile(jnp.transpose(token_table)).reshape(VOCAB, D)
    mesh = plsc.VectorSubcoreMesh(core_axis_name="c", subcore_axis_name="s")
    return pl.kernel(
        _emb_body,
        out_type=jax.ShapeDtypeStruct((L, DH, NW, DL, BW), jnp.float32),
        mesh=mesh,
        compiler_params=pltpu.CompilerParams(
            use_tc_tiling_on_sc=False, needs_layout_passes=False
        ),
        scratch_types=[
            pltpu.VMEM((NBUF, PL, BW), jnp.int32),
            pltpu.VMEM((NBUF, PL, BW, D), jnp.float32),
            pltpu.VMEM((NBUF, PL, D, BWP), jnp.float32),
            pltpu.VMEM((L, D), jnp.float32),
            pltpu.SemaphoreType.DMA((NBUF,)),
            pltpu.SemaphoreType.DMA((NBUF,)),
        ],
    )(x4, tlin, pos_table)


def kernel(x, token_table, pos_table):
    x4 = (
        x.astype(jnp.int32)
        .reshape(NW, BW, LH, LL)
        .transpose(2, 0, 3, 1)
    )
    out5 = _emb(x4, token_table, pos_table)
    return jnp.transpose(out5, (2, 4, 0, 1, 3)).reshape(B, L, D)


# speedup vs baseline: 2.1106x; 2.1106x over previous
import jax
import jax.numpy as jnp
from jax import lax
from jax.experimental import pallas as pl
from jax.experimental.pallas import tpu as pltpu
from jax.experimental.pallas import tpu_sc as plsc

VOCAB = 1_000_000
L = 200
D = 32
B = 4096

NC, NS = 2, 16
NW = NC * NS
BW = B // NW

LH, LL = L // 8, 8
DH, DL = D // 8, 8
PL = 4
G = L // PL
NBUF = 2
BWP = BW + 1


def _emb_body(x4_hbm, tok_hbm, pos_hbm, out_hbm, idx_v, rows_v, tblk_v, pos_v, gsem, wsem):
    cid = lax.axis_index("c")
    sid = lax.axis_index("s")
    wid = sid * NC + cid

    pltpu.sync_copy(pos_hbm, pos_v)

    iota = lax.iota(jnp.int32, 16)

    def fire(g):
        p = lax.rem(g, NBUF)
        lh = lax.div(g, LL // PL)
        ll0 = lax.rem(g, LL // PL) * PL
        pltpu.sync_copy(x4_hbm.at[lh, wid, pl.ds(ll0, PL)], idx_v.at[p])
        for j in range(PL):
            pltpu.async_copy(
                tok_hbm.at[idx_v.at[p, j]], rows_v.at[p, j], gsem.at[p]
            )

    fire(0)

    def chunk(g, _):
        p = lax.rem(g, NBUF)

        @pl.when(g + 1 < G)
        def _fire_next():
            fire(g + 1)

        for j in range(PL):
            pltpu.make_async_copy(
                tok_hbm.at[pl.ds(0, BW)], rows_v.at[p, j], gsem.at[p]
            ).wait()

        @pl.when(g >= NBUF)
        def _wait_writeback():
            for j in range(PL):
                for dh in range(DH):
                    pltpu.make_async_copy(
                        tblk_v.at[p, j, pl.ds(dh * DL, DL), pl.ds(0, BW)],
                        out_hbm.at[0, dh, 0],
                        wsem.at[p],
                    ).wait()

        for j in range(PL):
            lpos = g * PL + j
            prow0 = pos_v[lpos, pl.ds(0, 16)]
            prow1 = pos_v[lpos, pl.ds(16, 16)]
            rows = rows_v.at[p, j]
            tblk = tblk_v.at[p, j]

            @plsc.parallel_loop(0, BW, unroll=8)
            def _transpose(b):
                bsplat = jnp.broadcast_to(b, (16,))
                v0 = rows[b, pl.ds(0, 16)] + prow0
                v1 = rows[b, pl.ds(16, 16)] + prow1
                plsc.store_scatter(tblk, [iota, bsplat], v0)
                plsc.store_scatter(tblk, [iota + 16, bsplat], v1)

        for j in range(PL):
            for dh in range(DH):
                pltpu.async_copy(
                    tblk_v.at[p, j, pl.ds(dh * DL, DL), pl.ds(0, BW)],
                    out_hbm.at[g * PL + j, dh, wid],
                    wsem.at[p],
                )
        return 0

    lax.fori_loop(0, G, chunk, 0)

    for b in range(NBUF):
        for j in range(PL):
            for dh in range(DH):
                pltpu.make_async_copy(
                    tblk_v.at[b, j, pl.ds(dh * DL, DL), pl.ds(0, BW)],
                    out_hbm.at[0, dh, 0],
                    wsem.at[b],
                ).wait()


@jax.jit
def _emb(x4, token_table, pos_table):
    mesh = plsc.VectorSubcoreMesh(core_axis_name="c", subcore_axis_name="s")
    return pl.kernel(
        _emb_body,
        out_type=jax.ShapeDtypeStruct((L, DH, NW, DL, BW), jnp.float32),
        mesh=mesh,
        compiler_params=pltpu.CompilerParams(
            use_tc_tiling_on_sc=False, needs_layout_passes=False
        ),
        scratch_types=[
            pltpu.VMEM((NBUF, PL, BW), jnp.int32),
            pltpu.VMEM((NBUF, PL, BW, D), jnp.float32),
            pltpu.VMEM((NBUF, PL, D, BWP), jnp.float32),
            pltpu.VMEM((L, D), jnp.float32),
            pltpu.SemaphoreType.DMA((NBUF,)),
            pltpu.SemaphoreType.DMA((NBUF,)),
        ],
    )(x4, token_table, pos_table)


def kernel(x, token_table, pos_table):
    x4 = (
        x.astype(jnp.int32)
        .reshape(NW, BW, LH, LL)
        .transpose(2, 0, 3, 1)
    )
    out5 = _emb(x4, token_table, pos_table)
    return jnp.transpose(out5, (2, 4, 0, 1, 3)).reshape(B, L, D)
